# Initial kernel scaffold; baseline (speedup 1.0000x reference)
import jax
import jax.numpy as jnp
from jax.experimental import pallas as pl


def kernel(x, W, b, entries):
    raise NotImplementedError("write your pallas kernel here")


# speedup vs baseline: 5.5371x; 5.5371x over previous
import functools

import jax
import jax.numpy as jnp
from jax import lax
from jax.experimental import pallas as pl
from jax.experimental.pallas import tpu as pltpu
from jax.experimental.pallas import tpu_sc as plsc

G = 2
V = 320
GV = G * V
ENTRY_DIM = 128

_SC_CORES = 2
_SC_SUBCORES = 16
_NW = _SC_CORES * _SC_SUBCORES


def _tc_body(x_ref, wt_ref, b_ref, cb_ref, k0_ref, k1_ref, stats_ref,
             cnt_acc, soft_acc, *, n_rows):
    i = pl.program_id(0)

    @pl.when(i == 0)
    def _init():
        cnt_acc[...] = jnp.zeros_like(cnt_acc)
        soft_acc[...] = jnp.zeros_like(soft_acc)

    p = lax.dot_general(
        x_ref[...].astype(jnp.bfloat16), wt_ref[...].astype(jnp.bfloat16),
        (((1,), (0,)), ((), ())),
        preferred_element_type=jnp.float32,
    ) + b_ref[...]

    lane = lax.broadcasted_iota(jnp.int32, p.shape, 1)
    in0 = lane < V
    ninf = jnp.float32(-jnp.inf)
    m0 = jnp.max(jnp.where(in0, p, ninf), axis=1, keepdims=True)
    m1 = jnp.max(jnp.where(in0, ninf, p), axis=1, keepdims=True)
    k0 = jnp.min(jnp.where(in0 & (p == m0), lane, GV), axis=1, keepdims=True)
    k1 = jnp.min(jnp.where((~in0) & (p == m1), lane, GV), axis=1, keepdims=True)
    sel = jnp.where(in0, k0, k1)
    cb = (lane == sel).astype(jnp.float32)
    cb_ref[...] = cb
    k0_ref[...] = k0
    k1_ref[...] = k1

    m_sel = jnp.where(in0, m0, m1)
    e = jnp.exp(p - m_sel)
    s0 = jnp.sum(jnp.where(in0, e, 0.0), axis=1, keepdims=True)
    s1 = jnp.sum(jnp.where(in0, 0.0, e), axis=1, keepdims=True)
    sm = e / jnp.where(in0, s0, s1)

    cnt_acc[...] += jnp.sum(cb, axis=0, keepdims=True)
    soft_acc[...] += jnp.sum(sm, axis=0, keepdims=True)

    @pl.when(i == pl.num_programs(0) - 1)
    def _fini():
        lane1 = lax.broadcasted_iota(jnp.int32, (1, GV), 1)
        g0 = lane1 < V
        inv_n = jnp.float32(1.0 / n_rows)
        hp = cnt_acc[...] * inv_n
        t = hp * jnp.log(hp + 1e-7)
        code = (jnp.exp(-jnp.sum(jnp.where(g0, t, 0.0)))
                + jnp.exp(-jnp.sum(jnp.where(g0, 0.0, t))))
        q = soft_acc[...] * inv_n + 1e-7
        t2 = q * jnp.log(q + 1e-7)
        prob = (jnp.exp(-jnp.sum(jnp.where(g0, t2, 0.0)))
                + jnp.exp(-jnp.sum(jnp.where(g0, 0.0, t2))))
        stats_ref[0, 0] = code
        stats_ref[0, 1] = prob


def _tc_call(xf, Wt, b2, n_rows, blk):
    grid = (n_rows // blk,)
    return pl.pallas_call(
        functools.partial(_tc_body, n_rows=n_rows),
        grid=grid,
        in_specs=[
            pl.BlockSpec((blk, xf.shape[1]), lambda i: (i, 0)),
            pl.BlockSpec((Wt.shape[0], GV), lambda i: (0, 0)),
            pl.BlockSpec((1, GV), lambda i: (0, 0)),
        ],
        out_specs=[
            pl.BlockSpec((blk, GV), lambda i: (i, 0)),
            pl.BlockSpec((blk, 1), lambda i: (i, 0)),
            pl.BlockSpec((blk, 1), lambda i: (i, 0)),
            pl.BlockSpec(memory_space=pltpu.SMEM),
        ],
        out_shape=[
            jax.ShapeDtypeStruct((n_rows, GV), jnp.float32),
            jax.ShapeDtypeStruct((n_rows, 1), jnp.int32),
            jax.ShapeDtypeStruct((n_rows, 1), jnp.int32),
            jax.ShapeDtypeStruct((1, 2), jnp.float32),
        ],
        scratch_shapes=[
            pltpu.VMEM((1, GV), jnp.float32),
            pltpu.VMEM((1, GV), jnp.float32),
        ],
    )(xf, Wt, b2)


def _sc_gather(table, idx, n_idx):
    b_per_w = n_idx // _NW
    mesh = plsc.VectorSubcoreMesh(core_axis_name="c", subcore_axis_name="s")

    @functools.partial(
        pl.kernel,
        mesh=mesh,
        out_type=jax.ShapeDtypeStruct((n_idx, ENTRY_DIM), jnp.float32),
        scratch_types=[
            pltpu.VMEM((b_per_w,), jnp.int32),
            pltpu.VMEM((b_per_w, ENTRY_DIM), jnp.float32),
            pltpu.SemaphoreType.DMA,
        ],
    )
    def gather_k(table_hbm, idx_hbm, out_hbm, idx_v, rows_v, sem):
        wid = lax.axis_index("s") * _SC_CORES + lax.axis_index("c")
        base = wid * b_per_w
        pltpu.sync_copy(idx_hbm.at[pl.ds(base, b_per_w)], idx_v)
        pltpu.async_copy(table_hbm.at[idx_v], rows_v, sem).wait()
        pltpu.sync_copy(rows_v, out_hbm.at[pl.ds(base, b_per_w)])

    return gather_k(table, idx)


def kernel(x, W, b, entries):
    bsz, tsz, fsz = x.shape
    n_rows = bsz * tsz
    xf = x.reshape(n_rows, fsz)
    Wt = W.T
    b2 = b.reshape(1, GV)

    cb, k0, k1, stats = _tc_call(xf, Wt, b2, n_rows, blk=512)

    idx = jnp.concatenate([k0, k1], axis=1).reshape(n_rows * G)
    table = entries.reshape(GV, ENTRY_DIM)
    rows = _sc_gather(table, idx, n_rows * G)
    quantized = rows.reshape(bsz, tsz, G * ENTRY_DIM)

    return quantized, cb, stats[0, 0], stats[0, 1]


# speedup vs baseline: 8.3688x; 1.5114x over previous
import functools

import jax
import jax.numpy as jnp
from jax import lax
from jax.experimental import pallas as pl
from jax.experimental.pallas import tpu as pltpu
from jax.experimental.pallas import tpu_sc as plsc

G = 2
V = 320
GV = G * V
ENTRY_DIM = 128

_SC_CORES = 2
_SC_SUBCORES = 16
_NW = _SC_CORES * _SC_SUBCORES


def _tc_body(x_ref, wt_ref, b_ref, e_ref, cb_ref, k0_ref, k1_ref, q_ref, stats_ref,
             cnt_acc, soft_acc, *, n_rows):
    i = pl.program_id(0)

    @pl.when(i == 0)
    def _init():
        cnt_acc[...] = jnp.zeros_like(cnt_acc)
        soft_acc[...] = jnp.zeros_like(soft_acc)

    p = lax.dot_general(
        x_ref[...].astype(jnp.bfloat16), wt_ref[...].astype(jnp.bfloat16),
        (((1,), (0,)), ((), ())),
        preferred_element_type=jnp.float32,
    ) + b_ref[...]

    lane = lax.broadcasted_iota(jnp.int32, p.shape, 1)
    in0 = lane < V
    ninf = jnp.float32(-jnp.inf)
    m0 = jnp.max(jnp.where(in0, p, ninf), axis=1, keepdims=True)
    m1 = jnp.max(jnp.where(in0, ninf, p), axis=1, keepdims=True)
    k0 = jnp.min(jnp.where(in0 & (p == m0), lane, GV), axis=1, keepdims=True)
    k1 = jnp.min(jnp.where((~in0) & (p == m1), lane, GV), axis=1, keepdims=True)
    sel = jnp.where(in0, k0, k1)
    cb = (lane == sel).astype(jnp.float32)
    cb_ref[...] = cb
    k0_ref[...] = k0
    k1_ref[...] = k1
    q_ref[...] = lax.dot_general(
        cb, e_ref[...], (((1,), (0,)), ((), ())),
        preferred_element_type=jnp.float32,
        precision=lax.Precision.HIGHEST,
    )

    m_sel = jnp.where(in0, m0, m1)
    e = jnp.exp(p - m_sel)
    s0 = jnp.sum(jnp.where(in0, e, 0.0), axis=1, keepdims=True)
    s1 = jnp.sum(jnp.where(in0, 0.0, e), axis=1, keepdims=True)
    sm = e / jnp.where(in0, s0, s1)

    cnt_acc[...] += jnp.sum(cb, axis=0, keepdims=True)
    soft_acc[...] += jnp.sum(sm, axis=0, keepdims=True)

    @pl.when(i == pl.num_programs(0) - 1)
    def _fini():
        lane1 = lax.broadcasted_iota(jnp.int32, (1, GV), 1)
        g0 = lane1 < V
        inv_n = jnp.float32(1.0 / n_rows)
        hp = cnt_acc[...] * inv_n
        t = hp * jnp.log(hp + 1e-7)
        code = (jnp.exp(-jnp.sum(jnp.where(g0, t, 0.0)))
                + jnp.exp(-jnp.sum(jnp.where(g0, 0.0, t))))
        q = soft_acc[...] * inv_n + 1e-7
        t2 = q * jnp.log(q + 1e-7)
        prob = (jnp.exp(-jnp.sum(jnp.where(g0, t2, 0.0)))
                + jnp.exp(-jnp.sum(jnp.where(g0, 0.0, t2))))
        stats_ref[0, 0] = code
        stats_ref[0, 1] = prob


def _tc_call(xf, Wt, b2, E, n_rows, blk):
    grid = (n_rows // blk,)
    return pl.pallas_call(
        functools.partial(_tc_body, n_rows=n_rows),
        grid=grid,
        in_specs=[
            pl.BlockSpec((blk, xf.shape[1]), lambda i: (i, 0)),
            pl.BlockSpec((Wt.shape[0], GV), lambda i: (0, 0)),
            pl.BlockSpec((1, GV), lambda i: (0, 0)),
            pl.BlockSpec((GV, G * ENTRY_DIM), lambda i: (0, 0)),
        ],
        out_specs=[
            pl.BlockSpec((blk, GV), lambda i: (i, 0)),
            pl.BlockSpec((blk, 1), lambda i: (i, 0)),
            pl.BlockSpec((blk, 1), lambda i: (i, 0)),
            pl.BlockSpec((blk, G * ENTRY_DIM), lambda i: (i, 0)),
            pl.BlockSpec(memory_space=pltpu.SMEM),
        ],
        out_shape=[
            jax.ShapeDtypeStruct((n_rows, GV), jnp.float32),
            jax.ShapeDtypeStruct((n_rows, 1), jnp.int32),
            jax.ShapeDtypeStruct((n_rows, 1), jnp.int32),
            jax.ShapeDtypeStruct((n_rows, G * ENTRY_DIM), jnp.float32),
            jax.ShapeDtypeStruct((1, 2), jnp.float32),
        ],
        scratch_shapes=[
            pltpu.VMEM((1, GV), jnp.float32),
            pltpu.VMEM((1, GV), jnp.float32),
        ],
    )(xf, Wt, b2, E)


def _sc_gather(table, idx, n_idx):
    b_per_w = n_idx // _NW
    mesh = plsc.VectorSubcoreMesh(core_axis_name="c", subcore_axis_name="s")

    @functools.partial(
        pl.kernel,
        mesh=mesh,
        out_type=jax.ShapeDtypeStruct((n_idx, ENTRY_DIM), jnp.float32),
        scratch_types=[
            pltpu.VMEM((b_per_w,), jnp.int32),
            pltpu.VMEM((b_per_w, ENTRY_DIM), jnp.float32),
            pltpu.SemaphoreType.DMA,
        ],
    )
    def gather_k(table_hbm, idx_hbm, out_hbm, idx_v, rows_v, sem):
        wid = lax.axis_index("s") * _SC_CORES + lax.axis_index("c")
        base = wid * b_per_w
        pltpu.sync_copy(idx_hbm.at[pl.ds(base, b_per_w)], idx_v)
        pltpu.async_copy(table_hbm.at[idx_v], rows_v, sem).wait()
        pltpu.sync_copy(rows_v, out_hbm.at[pl.ds(base, b_per_w)])

    return gather_k(table, idx)


def kernel(x, W, b, entries):
    bsz, tsz, fsz = x.shape
    n_rows = bsz * tsz
    xf = x.reshape(n_rows, fsz)
    Wt = W.T
    b2 = b.reshape(1, GV)

    ent = entries.reshape(GV, ENTRY_DIM)
    E = jnp.zeros((GV, G * ENTRY_DIM), jnp.float32)
    E = E.at[:V, :ENTRY_DIM].set(ent[:V])
    E = E.at[V:, ENTRY_DIM:].set(ent[V:])

    cb, k0, k1, q, stats = _tc_call(xf, Wt, b2, E, n_rows, blk=512)
    quantized = q.reshape(bsz, tsz, G * ENTRY_DIM)

    return quantized, cb, stats[0, 0], stats[0, 1]
